# Initial kernel scaffold; baseline (speedup 1.0000x reference)
import jax
import jax.numpy as jnp
from jax.experimental import pallas as pl


def kernel(x, edge_index, W1, b1, W2, b2, W3, b3, A1w, A1b, A2w, A2b, A3w, A3b, C1w, C1b, C2w, C2b, C3w, C3b):
    raise NotImplementedError("write your pallas kernel here")


# speedup vs baseline: 10.0607x; 10.0607x over previous
import functools

import jax
import jax.numpy as jnp
from jax import lax
from jax.experimental import pallas as pl
from jax.experimental.pallas import tpu as pltpu, tpu_sc as plsc

N = 10000
E = 320000
D = 128
H = 128
OUT = 8

NC = 2
NS = 16
NW = NC * NS
E_W = E // NW
K = 80
ITERS = E_W // K
BLK = 1000


def _deg_body(dst_hbm, ones_hbm, zero_hbm, out_hbm, idx_v, ones_v, acc, sem):
    cid = lax.axis_index("c")
    sid = lax.axis_index("s")
    wid = sid * NC + cid

    @pl.when(sid == 0)
    def _():
        pltpu.sync_copy(zero_hbm, acc)

    plsc.subcore_barrier()
    pltpu.sync_copy(ones_hbm, ones_v)
    base = wid * E_W

    def body(i, carry):
        off = base + i * K
        pltpu.sync_copy(dst_hbm.at[pl.ds(off, K)], idx_v)
        pltpu.sync_copy(ones_v, acc.at[idx_v], add=True)
        return carry

    lax.fori_loop(0, ITERS, body, 0)
    plsc.subcore_barrier()

    @pl.when(sid == 0)
    def _():
        pltpu.sync_copy(acc, out_hbm.at[cid])


def _degree_partials(dst):
    mesh = plsc.VectorSubcoreMesh(core_axis_name="c", subcore_axis_name="s")
    ones = jnp.ones((K, H), jnp.float32)
    zero = jnp.zeros((N, H), jnp.float32)
    f = pl.kernel(
        _deg_body,
        out_type=jax.ShapeDtypeStruct((NC, N, H), jnp.float32),
        mesh=mesh,
        scratch_types=[
            pltpu.VMEM((K,), jnp.int32),
            pltpu.VMEM((K, H), jnp.float32),
            pltpu.VMEM_SHARED((N, H), jnp.float32),
            pltpu.SemaphoreType.DMA,
        ],
    )
    return f(dst, ones, zero)


def _agg_body(gp_hbm, src_hbm, dst_hbm, zero_hbm, out_hbm,
              idx_s, idx_d, rows, acc, sem):
    cid = lax.axis_index("c")
    sid = lax.axis_index("s")
    wid = sid * NC + cid

    @pl.when(sid == 0)
    def _():
        pltpu.sync_copy(zero_hbm, acc)

    plsc.subcore_barrier()
    base = wid * E_W

    def body(i, carry):
        off = base + i * K
        pltpu.sync_copy(src_hbm.at[pl.ds(off, K)], idx_s)
        pltpu.sync_copy(dst_hbm.at[pl.ds(off, K)], idx_d)
        pltpu.async_copy(gp_hbm.at[idx_s], rows, sem).wait()
        pltpu.sync_copy(rows, acc.at[idx_d], add=True)
        return carry

    lax.fori_loop(0, ITERS, body, 0)
    plsc.subcore_barrier()

    @pl.when(sid == 0)
    def _():
        pltpu.sync_copy(acc, out_hbm.at[cid])


def _edge_aggregate(gp, src, dst):
    mesh = plsc.VectorSubcoreMesh(core_axis_name="c", subcore_axis_name="s")
    zero = jnp.zeros((N, H), jnp.float32)
    f = pl.kernel(
        _agg_body,
        out_type=jax.ShapeDtypeStruct((NC, N, H), jnp.float32),
        mesh=mesh,
        scratch_types=[
            pltpu.VMEM((K,), jnp.int32),
            pltpu.VMEM((K,), jnp.int32),
            pltpu.VMEM((K, H), jnp.float32),
            pltpu.VMEM_SHARED((N, H), jnp.float32),
            pltpu.SemaphoreType.DMA,
        ],
    )
    return f(gp, src, dst, zero)


def _mm_body(x_ref, w_ref, o_ref):
    o_ref[...] = jnp.dot(x_ref[...], w_ref[...],
                         preferred_element_type=jnp.float32)


def _matmul(x, w):
    m, k = x.shape
    _, n = w.shape
    return pl.pallas_call(
        _mm_body,
        grid=(m // BLK,),
        in_specs=[
            pl.BlockSpec((BLK, k), lambda i: (i, 0)),
            pl.BlockSpec((k, n), lambda i: (0, 0)),
        ],
        out_specs=pl.BlockSpec((BLK, n), lambda i: (i, 0)),
        out_shape=jax.ShapeDtypeStruct((m, n), jnp.float32),
    )(x, w)


def _dinv_scale_body(p0_ref, p1_ref, m_ref, dinv_ref, g_ref):
    deg = p0_ref[:, 0:1] + p1_ref[:, 0:1] + 1.0
    dinv = lax.rsqrt(deg)
    dinv_b = jnp.broadcast_to(dinv, m_ref.shape)
    dinv_ref[...] = dinv_b
    g_ref[...] = dinv_b * m_ref[...]


def _dinv_and_scale(degp, m1):
    return pl.pallas_call(
        _dinv_scale_body,
        grid=(N // BLK,),
        in_specs=[
            pl.BlockSpec((BLK, H), lambda i: (i, 0)),
            pl.BlockSpec((BLK, H), lambda i: (i, 0)),
            pl.BlockSpec((BLK, H), lambda i: (i, 0)),
        ],
        out_specs=[
            pl.BlockSpec((BLK, H), lambda i: (i, 0)),
            pl.BlockSpec((BLK, H), lambda i: (i, 0)),
        ],
        out_shape=[
            jax.ShapeDtypeStruct((N, H), jnp.float32),
            jax.ShapeDtypeStruct((N, H), jnp.float32),
        ],
    )(degp[0], degp[1], m1)


def _layer_body(g_ref, p0_ref, p1_ref, dinv_ref, b_ref, w_ref, o_ref):
    dinv = dinv_ref[...]
    h = jnp.maximum(dinv * (g_ref[...] + p0_ref[...] + p1_ref[...])
                    + b_ref[...], 0.0)
    o_ref[...] = dinv * jnp.dot(h, w_ref[...],
                                preferred_element_type=jnp.float32)


def _layer(gp, parts, dinv, b, w):
    return pl.pallas_call(
        _layer_body,
        grid=(N // BLK,),
        in_specs=[
            pl.BlockSpec((BLK, H), lambda i: (i, 0)),
            pl.BlockSpec((BLK, H), lambda i: (i, 0)),
            pl.BlockSpec((BLK, H), lambda i: (i, 0)),
            pl.BlockSpec((BLK, H), lambda i: (i, 0)),
            pl.BlockSpec((1, H), lambda i: (0, 0)),
            pl.BlockSpec((H, H), lambda i: (0, 0)),
        ],
        out_specs=pl.BlockSpec((BLK, H), lambda i: (i, 0)),
        out_shape=jax.ShapeDtypeStruct((N, H), jnp.float32),
    )(gp, parts[0], parts[1], dinv, b.reshape(1, H), w)


def _heads_body(g_ref, p0_ref, p1_ref, dinv_ref, b_ref,
                a1w_ref, a1b_ref, a2w_ref, a2b_ref, a3w_ref, a3b_ref,
                c1w_ref, c1b_ref, c2w_ref, c2b_ref, c3w_ref, c3b_ref,
                probs_ref, vals_ref):
    dinv = dinv_ref[...]
    h = jnp.maximum(dinv * (g_ref[...] + p0_ref[...] + p1_ref[...])
                    + b_ref[...], 0.0)
    a = jnp.maximum(jnp.dot(h, a1w_ref[...],
                            preferred_element_type=jnp.float32)
                    + a1b_ref[...], 0.0)
    a = jnp.maximum(jnp.dot(a, a2w_ref[...],
                            preferred_element_type=jnp.float32)
                    + a2b_ref[...], 0.0)
    logits = jnp.dot(a, a3w_ref[...],
                     preferred_element_type=jnp.float32) + a3b_ref[...]
    m = jnp.max(logits, axis=-1, keepdims=True)
    e = jnp.exp(logits - m)
    probs_ref[...] = e / jnp.sum(e, axis=-1, keepdims=True)
    c = jnp.maximum(jnp.dot(h, c1w_ref[...],
                            preferred_element_type=jnp.float32)
                    + c1b_ref[...], 0.0)
    c = jnp.maximum(jnp.dot(c, c2w_ref[...],
                            preferred_element_type=jnp.float32)
                    + c2b_ref[...], 0.0)
    vals_ref[...] = jnp.dot(c, c3w_ref[...],
                            preferred_element_type=jnp.float32) + c3b_ref[...]


def _heads(gp, parts, dinv, b3, A1w, A1b, A2w, A2b, A3w, A3b,
           C1w, C1b, C2w, C2b, C3w, C3b):
    full = lambda r, c: pl.BlockSpec((r, c), lambda i: (0, 0))
    row = lambda c: pl.BlockSpec((BLK, c), lambda i: (i, 0))
    return pl.pallas_call(
        _heads_body,
        grid=(N // BLK,),
        in_specs=[
            row(H), row(H), row(H), row(H), full(1, H),
            full(H, 2 * H), full(1, 2 * H),
            full(2 * H, H), full(1, H),
            full(H, OUT), full(1, OUT),
            full(H, 2 * H), full(1, 2 * H),
            full(2 * H, H), full(1, H),
            full(H, 1), full(1, 1),
        ],
        out_specs=[row(OUT), row(1)],
        out_shape=[
            jax.ShapeDtypeStruct((N, OUT), jnp.float32),
            jax.ShapeDtypeStruct((N, 1), jnp.float32),
        ],
    )(gp, parts[0], parts[1], dinv, b3.reshape(1, H),
      A1w, A1b.reshape(1, 2 * H), A2w, A2b.reshape(1, H),
      A3w, A3b.reshape(1, OUT),
      C1w, C1b.reshape(1, 2 * H), C2w, C2b.reshape(1, H),
      C3w, C3b.reshape(1, 1))


def kernel(x, edge_index, W1, b1, W2, b2, W3, b3,
           A1w, A1b, A2w, A2b, A3w, A3b,
           C1w, C1b, C2w, C2b, C3w, C3b):
    src = edge_index[0]
    dst = edge_index[1]

    m1 = _matmul(x, W1)
    degp = _degree_partials(dst)
    dinv, g1 = _dinv_and_scale(degp, m1)

    p1 = _edge_aggregate(g1, src, dst)
    g2 = _layer(g1, p1, dinv, b1, W2)
    p2 = _edge_aggregate(g2, src, dst)
    g3 = _layer(g2, p2, dinv, b2, W3)
    p3 = _edge_aggregate(g3, src, dst)

    probs, vals = _heads(g3, p3, dinv, b3, A1w, A1b, A2w, A2b, A3w, A3b,
                         C1w, C1b, C2w, C2b, C3w, C3b)
    return (probs, vals)
